# Initial kernel scaffold; baseline (speedup 1.0000x reference)
import jax
import jax.numpy as jnp
from jax.experimental import pallas as pl


def kernel(X_in, ones):
    raise NotImplementedError("write your pallas kernel here")


# speedup vs baseline: 1.0968x; 1.0968x over previous
import functools

import jax
import jax.numpy as jnp
from jax import lax
from jax.experimental import pallas as pl
from jax.experimental.pallas import tpu as pltpu
from jax.experimental.pallas import tpu_sc as plsc

_DEPTH = 1000
_BATCH = 16384
_NUM_CORES = 2
_NUM_SUBCORES = 16
_NUM_WORKERS = _NUM_CORES * _NUM_SUBCORES
_ROWS_PER_WORKER = _BATCH // _NUM_WORKERS
_CHUNK_ROWS = 64
_NUM_CHUNKS = _ROWS_PER_WORKER // _CHUNK_ROWS
_CHUNK_WORDS = _CHUNK_ROWS * _DEPTH
_LANES = 16


def _onehot_body(x_hbm, out_hbm, idx_v, buf0, buf1, sem0, sem1):
    wid = lax.axis_index("s") * _NUM_CORES + lax.axis_index("c")
    row_base = wid * _ROWS_PER_WORKER
    elem_base = row_base * _DEPTH

    pltpu.sync_copy(x_hbm.at[pl.ds(row_base, _ROWS_PER_WORKER)], idx_v)

    lane = lax.iota(jnp.int32, _LANES)
    ones_v = jnp.full((_LANES,), 1.0, jnp.float32)
    zeros_v = jnp.zeros((_LANES,), jnp.float32)

    def zero_buf(buf):
        unroll = 8
        span = unroll * _LANES

        def body(i, carry):
            start = i * span
            for k in range(unroll):
                buf[pl.ds(start + k * _LANES, _LANES)] = zeros_v
            return carry

        lax.fori_loop(0, _CHUNK_WORDS // span, body, 0)

    def scatter_chunk(buf, c, vals):
        for g in range(_CHUNK_ROWS // _LANES):
            cols = idx_v[pl.ds(c * _CHUNK_ROWS + g * _LANES, _LANES)]
            flat = (lane + g * _LANES) * _DEPTH + cols
            plsc.store_scatter(buf, [flat], vals)

    def chunk_dma(buf, c, sem):
        dst = out_hbm.at[pl.ds(elem_base + c * _CHUNK_WORDS, _CHUNK_WORDS)]
        return pltpu.make_async_copy(buf, dst, sem)

    for c in range(_NUM_CHUNKS):
        buf = buf0 if c % 2 == 0 else buf1
        sem = sem0 if c % 2 == 0 else sem1
        if c < 2:
            zero_buf(buf)
        else:
            chunk_dma(buf, c - 2, sem).wait()
            scatter_chunk(buf, c - 2, zeros_v)
        scatter_chunk(buf, c, ones_v)
        chunk_dma(buf, c, sem).start()

    chunk_dma(buf0, _NUM_CHUNKS - 2, sem0).wait()
    chunk_dma(buf1, _NUM_CHUNKS - 1, sem1).wait()


@jax.jit
def _onehot_sc(x):
    mesh = plsc.VectorSubcoreMesh(core_axis_name="c", subcore_axis_name="s")
    fn = functools.partial(
        pl.kernel,
        mesh=mesh,
        out_type=jax.ShapeDtypeStruct((_BATCH * _DEPTH,), jnp.float32),
        scratch_types=[
            pltpu.VMEM((_ROWS_PER_WORKER,), jnp.int32),
            pltpu.VMEM((_CHUNK_WORDS,), jnp.float32),
            pltpu.VMEM((_CHUNK_WORDS,), jnp.float32),
            pltpu.SemaphoreType.DMA,
            pltpu.SemaphoreType.DMA,
        ],
        compiler_params=pltpu.CompilerParams(needs_layout_passes=False),
    )(_onehot_body)
    return fn(x)


def kernel(X_in, ones):
    del ones
    flat = _onehot_sc(X_in.astype(jnp.int32))
    return flat.reshape(_BATCH, _DEPTH)


# speedup vs baseline: 1.7830x; 1.6257x over previous
import functools

import jax
import jax.numpy as jnp
from jax import lax
from jax.experimental import pallas as pl
from jax.experimental.pallas import tpu as pltpu
from jax.experimental.pallas import tpu_sc as plsc

_DEPTH = 1000
_BATCH = 16384
_NUM_CORES = 2
_NUM_SUBCORES = 16
_NUM_WORKERS = _NUM_CORES * _NUM_SUBCORES
_ROWS_PER_WORKER = _BATCH // _NUM_WORKERS
_CHUNK_ROWS = 32
_NUM_CHUNKS = _ROWS_PER_WORKER // _CHUNK_ROWS
_LANES = 16
_FULL_SLICES = _DEPTH // _LANES
_REM_COLS = _DEPTH - _FULL_SLICES * _LANES


def _onehot_body(x_hbm, out_hbm, idx_v, buf0, buf1, sem0, sem1):
    wid = lax.axis_index("s") * _NUM_CORES + lax.axis_index("c")
    row_base = wid * _ROWS_PER_WORKER

    pltpu.sync_copy(x_hbm.at[pl.ds(row_base, _ROWS_PER_WORKER)], idx_v)

    lane = lax.iota(jnp.int32, _LANES)
    ones_v = jnp.full((_LANES,), 1.0, jnp.float32)
    zeros_v = jnp.zeros((_LANES,), jnp.float32)

    def zero_buf(buf):
        def body(r, carry):
            for k in range(_FULL_SLICES):
                buf[r, pl.ds(k * _LANES, _LANES)] = zeros_v
            return carry

        lax.fori_loop(0, _CHUNK_ROWS, body, 0)
        rem_row = lane // _REM_COLS
        rem_col = _FULL_SLICES * _LANES + (lane % _REM_COLS)
        for m in range(_CHUNK_ROWS // 2):
            plsc.store_scatter(buf, [rem_row + 2 * m, rem_col], zeros_v)

    def scatter_chunk(buf, c, vals):
        for g in range(_CHUNK_ROWS // _LANES):
            cols = idx_v[pl.ds(c * _CHUNK_ROWS + g * _LANES, _LANES)]
            plsc.store_scatter(buf, [lane + g * _LANES, cols], vals)

    def chunk_dma(buf, c, sem):
        dst = out_hbm.at[pl.ds(row_base + c * _CHUNK_ROWS, _CHUNK_ROWS), :]
        return pltpu.make_async_copy(buf, dst, sem)

    for c in range(_NUM_CHUNKS):
        buf = buf0 if c % 2 == 0 else buf1
        sem = sem0 if c % 2 == 0 else sem1
        if c < 2:
            zero_buf(buf)
        else:
            chunk_dma(buf, c - 2, sem).wait()
            scatter_chunk(buf, c - 2, zeros_v)
        scatter_chunk(buf, c, ones_v)
        chunk_dma(buf, c, sem).start()

    chunk_dma(buf0, _NUM_CHUNKS - 2, sem0).wait()
    chunk_dma(buf1, _NUM_CHUNKS - 1, sem1).wait()


@jax.jit
def _onehot_sc(x):
    mesh = plsc.VectorSubcoreMesh(core_axis_name="c", subcore_axis_name="s")
    fn = functools.partial(
        pl.kernel,
        mesh=mesh,
        out_type=jax.ShapeDtypeStruct((_BATCH, _DEPTH), jnp.float32),
        scratch_types=[
            pltpu.VMEM((_ROWS_PER_WORKER,), jnp.int32),
            pltpu.VMEM((_CHUNK_ROWS, _DEPTH), jnp.float32),
            pltpu.VMEM((_CHUNK_ROWS, _DEPTH), jnp.float32),
            pltpu.SemaphoreType.DMA,
            pltpu.SemaphoreType.DMA,
        ],
        compiler_params=pltpu.CompilerParams(needs_layout_passes=False),
    )(_onehot_body)
    return fn(x)


def kernel(X_in, ones):
    del ones
    return _onehot_sc(X_in.astype(jnp.int32))


# speedup vs baseline: 4.1552x; 2.3305x over previous
import functools

import jax
import jax.numpy as jnp
from jax import lax
from jax.experimental import pallas as pl
from jax.experimental.pallas import tpu as pltpu
from jax.experimental.pallas import tpu_sc as plsc

_DEPTH = 1000
_BATCH = 16384
_NUM_CORES = 2
_NUM_SUBCORES = 16
_NUM_WORKERS = _NUM_CORES * _NUM_SUBCORES
_COLS_PER_WORKER = _BATCH // _NUM_WORKERS
_CHUNK_D = 104
_LANES = 16
_CHUNK_STARTS = tuple(range(0, _DEPTH - 64, _CHUNK_D)) + (_DEPTH - 64,)
_CHUNK_LENS = (_CHUNK_D,) * (len(_CHUNK_STARTS) - 1) + (64,)
_NUM_CHUNKS = len(_CHUNK_STARTS)


def _onehot_body(x_hbm, out_hbm, idx_v, buf0, buf1, sem0, sem1):
    wid = lax.axis_index("s") * _NUM_CORES + lax.axis_index("c")
    col_base = wid * _COLS_PER_WORKER

    pltpu.sync_copy(x_hbm.at[pl.ds(col_base, _COLS_PER_WORKER)], idx_v)

    lane = lax.iota(jnp.int32, _LANES)
    ones_v = jnp.full((_LANES,), 1.0, jnp.float32)
    zeros_v = jnp.zeros((_LANES,), jnp.float32)

    def zero_buf(buf):
        def body(r, carry):
            for k in range(_COLS_PER_WORKER // _LANES):
                buf[r, pl.ds(k * _LANES, _LANES)] = zeros_v
            return carry

        lax.fori_loop(0, _CHUNK_D, body, 0)

    def scatter_chunk(buf, c, vals):
        d0 = _CHUNK_STARTS[c]
        d1 = d0 + _CHUNK_LENS[c]

        def body(i, carry):
            idx = idx_v[pl.ds(i * _LANES, _LANES)]
            mask = (idx >= d0) & (idx < d1)
            cols = lane + i * _LANES
            plsc.store_scatter(buf, [idx - d0, cols], vals, mask=mask)
            return carry

        lax.fori_loop(0, _COLS_PER_WORKER // _LANES, body, 0)

    def chunk_dma(buf, c, sem):
        d0, dlen = _CHUNK_STARTS[c], _CHUNK_LENS[c]
        dst = out_hbm.at[pl.ds(d0, dlen), pl.ds(col_base, _COLS_PER_WORKER)]
        src = buf if dlen == _CHUNK_D else buf.at[pl.ds(0, dlen), :]
        return pltpu.make_async_copy(src, dst, sem)

    for c in range(_NUM_CHUNKS):
        buf = buf0 if c % 2 == 0 else buf1
        sem = sem0 if c % 2 == 0 else sem1
        if c < 2:
            zero_buf(buf)
        else:
            chunk_dma(buf, c - 2, sem).wait()
            scatter_chunk(buf, c - 2, zeros_v)
        scatter_chunk(buf, c, ones_v)
        chunk_dma(buf, c, sem).start()

    chunk_dma(buf0, _NUM_CHUNKS - 2, sem0).wait()
    chunk_dma(buf1, _NUM_CHUNKS - 1, sem1).wait()


@jax.jit
def _onehot_sc(x):
    mesh = plsc.VectorSubcoreMesh(core_axis_name="c", subcore_axis_name="s")
    fn = functools.partial(
        pl.kernel,
        mesh=mesh,
        out_type=jax.ShapeDtypeStruct((_DEPTH, _BATCH), jnp.float32),
        scratch_types=[
            pltpu.VMEM((_COLS_PER_WORKER,), jnp.int32),
            pltpu.VMEM((_CHUNK_D, _COLS_PER_WORKER), jnp.float32),
            pltpu.VMEM((_CHUNK_D, _COLS_PER_WORKER), jnp.float32),
            pltpu.SemaphoreType.DMA,
            pltpu.SemaphoreType.DMA,
        ],
        compiler_params=pltpu.CompilerParams(needs_layout_passes=False),
    )(_onehot_body)
    return fn(x)


def kernel(X_in, ones):
    del ones
    return _onehot_sc(X_in.astype(jnp.int32)).T


# speedup vs baseline: 4.1574x; 1.0005x over previous
import functools

import jax
import jax.numpy as jnp
from jax import lax
from jax.experimental import pallas as pl
from jax.experimental.pallas import tpu as pltpu
from jax.experimental.pallas import tpu_sc as plsc

_DEPTH = 1000
_BATCH = 16384
_NUM_CORES = 2
_NUM_SUBCORES = 16
_NUM_WORKERS = _NUM_CORES * _NUM_SUBCORES
_COLS_PER_WORKER = _BATCH // _NUM_WORKERS
_CHUNK_D = 104
_LANES = 16
_CHUNK_STARTS = tuple(range(0, _DEPTH - 64, _CHUNK_D)) + (_DEPTH - 64,)
_CHUNK_LENS = (_CHUNK_D,) * (len(_CHUNK_STARTS) - 1) + (64,)
_NUM_CHUNKS = len(_CHUNK_STARTS)


def _onehot_body(x_hbm, out_hbm, idx_v, buf0, buf1, sem0, sem1):
    wid = lax.axis_index("s") * _NUM_CORES + lax.axis_index("c")
    col_base = wid * _COLS_PER_WORKER

    pltpu.sync_copy(x_hbm.at[pl.ds(col_base, _COLS_PER_WORKER)], idx_v)

    lane = lax.iota(jnp.int32, _LANES)
    ones_v = jnp.full((_LANES,), 1.0, jnp.float32)
    zeros_v = jnp.zeros((_LANES,), jnp.float32)

    def zero_buf(buf):
        def body(r, carry):
            for k in range(_COLS_PER_WORKER // _LANES):
                buf[r, pl.ds(k * _LANES, _LANES)] = zeros_v
            return carry

        lax.fori_loop(0, _CHUNK_D, body, 0)

    def scatter_chunk(buf, c, vals):
        d0 = _CHUNK_STARTS[c]
        d1 = d0 + _CHUNK_LENS[c]

        def body(i, carry):
            idx = idx_v[pl.ds(i * _LANES, _LANES)]
            mask = (idx >= d0) & (idx < d1)
            cols = lane + i * _LANES
            plsc.store_scatter(buf, [idx - d0, cols], vals, mask=mask)
            return carry

        lax.fori_loop(0, _COLS_PER_WORKER // _LANES, body, 0)

    def chunk_dma(buf, c, sem):
        d0, dlen = _CHUNK_STARTS[c], _CHUNK_LENS[c]
        dst = out_hbm.at[pl.ds(d0, dlen), pl.ds(col_base, _COLS_PER_WORKER)]
        src = buf if dlen == _CHUNK_D else buf.at[pl.ds(0, dlen), :]
        return pltpu.make_async_copy(src, dst, sem)

    for c in range(_NUM_CHUNKS):
        buf = buf0 if c % 2 == 0 else buf1
        sem = sem0 if c % 2 == 0 else sem1
        if c < 2:
            zero_buf(buf)
        else:
            chunk_dma(buf, c - 2, sem).wait()
            scatter_chunk(buf, c - 2, zeros_v)
        scatter_chunk(buf, c, ones_v)
        chunk_dma(buf, c, sem).start()

    chunk_dma(buf0, _NUM_CHUNKS - 2, sem0).wait()
    chunk_dma(buf1, _NUM_CHUNKS - 1, sem1).wait()


@jax.jit
def _onehot_sc(x):
    mesh = plsc.VectorSubcoreMesh(core_axis_name="c", subcore_axis_name="s")
    fn = functools.partial(
        pl.kernel,
        mesh=mesh,
        out_type=jax.ShapeDtypeStruct((_DEPTH, _BATCH), jnp.float32),
        scratch_types=[
            pltpu.VMEM((_COLS_PER_WORKER,), jnp.int32),
            pltpu.VMEM((_CHUNK_D, _COLS_PER_WORKER), jnp.float32),
            pltpu.VMEM((_CHUNK_D, _COLS_PER_WORKER), jnp.float32),
            pltpu.SemaphoreType.DMA,
            pltpu.SemaphoreType.DMA,
        ],
        compiler_params=pltpu.CompilerParams(
            needs_layout_passes=False,
            skip_device_barrier=True,
            disable_semaphore_checks=True,
        ),
    )(_onehot_body)
    return fn(x)


def kernel(X_in, ones):
    del ones
    return _onehot_sc(X_in.astype(jnp.int32)).T


# speedup vs baseline: 4.1640x; 1.0016x over previous
import functools

import jax
import jax.numpy as jnp
from jax import lax
from jax.experimental import pallas as pl
from jax.experimental.pallas import tpu as pltpu
from jax.experimental.pallas import tpu_sc as plsc

_DEPTH = 1000
_BATCH = 16384
_NUM_CORES = 2
_NUM_SUBCORES = 16
_NUM_WORKERS = _NUM_CORES * _NUM_SUBCORES
_COLS_PER_WORKER = _BATCH // _NUM_WORKERS
_CHUNK_D = 120
_TAIL_D = _DEPTH - 8 * _CHUNK_D
_LANES = 16
_SCANS = _COLS_PER_WORKER // _LANES


def _onehot_body(x_hbm, out_hbm, idx_v, buf0, buf1, sem0, sem1):
    wid = lax.axis_index("s") * _NUM_CORES + lax.axis_index("c")
    col_base = wid * _COLS_PER_WORKER

    pltpu.sync_copy(x_hbm.at[pl.ds(col_base, _COLS_PER_WORKER)], idx_v)

    lane = lax.iota(jnp.int32, _LANES)
    ones_v = jnp.full((_LANES,), 1.0, jnp.float32)
    zeros_v = jnp.zeros((_LANES,), jnp.float32)

    def zero_buf(buf):
        def body(r, carry):
            for k in range(_COLS_PER_WORKER // _LANES):
                buf[r, pl.ds(k * _LANES, _LANES)] = zeros_v
            return carry

        lax.fori_loop(0, _CHUNK_D, body, 0)

    def scan_pass(buf, d0_set, set_len, d0_clear):
        def body(i, carry):
            idx = idx_v[pl.ds(i * _LANES, _LANES)]
            cols = lane + i * _LANES
            if d0_clear is not None:
                mc = (idx >= d0_clear) & (idx < d0_clear + _CHUNK_D)
                plsc.store_scatter(buf, [idx - d0_clear, cols], zeros_v, mask=mc)
            ms = (idx >= d0_set) & (idx < d0_set + set_len)
            plsc.store_scatter(buf, [idx - d0_set, cols], ones_v, mask=ms)
            return carry

        lax.fori_loop(0, _SCANS, body, 0)

    def chunk_dma(buf, d0, dlen, sem):
        dst = out_hbm.at[pl.ds(d0, dlen), pl.ds(col_base, _COLS_PER_WORKER)]
        src = buf if dlen == _CHUNK_D else buf.at[pl.ds(0, dlen), :]
        return pltpu.make_async_copy(src, dst, sem)

    zero_buf(buf0)
    scan_pass(buf0, 0, _CHUNK_D, None)
    chunk_dma(buf0, 0, _CHUNK_D, sem0).start()
    zero_buf(buf1)
    scan_pass(buf1, _CHUNK_D, _CHUNK_D, None)
    chunk_dma(buf1, _CHUNK_D, _CHUNK_D, sem1).start()

    def pair(p, carry):
        base = pl.multiple_of(p * 2 * _CHUNK_D, 8)
        for buf, sem, d0 in ((buf0, sem0, base), (buf1, sem1, base + _CHUNK_D)):
            chunk_dma(buf, d0, _CHUNK_D, sem).wait()
            scan_pass(buf, d0, _CHUNK_D, d0 - 2 * _CHUNK_D)
            chunk_dma(buf, d0, _CHUNK_D, sem).start()
        return carry

    lax.fori_loop(1, 4, pair, 0)

    d6, d8 = 6 * _CHUNK_D, 8 * _CHUNK_D
    chunk_dma(buf0, d6, _CHUNK_D, sem0).wait()
    scan_pass(buf0, d8, _TAIL_D, d6)
    chunk_dma(buf0, d8, _TAIL_D, sem0).start()

    chunk_dma(buf1, 7 * _CHUNK_D, _CHUNK_D, sem1).wait()
    chunk_dma(buf0, d8, _TAIL_D, sem0).wait()


@jax.jit
def _onehot_sc(x):
    mesh = plsc.VectorSubcoreMesh(core_axis_name="c", subcore_axis_name="s")
    fn = functools.partial(
        pl.kernel,
        mesh=mesh,
        out_type=jax.ShapeDtypeStruct((_DEPTH, _BATCH), jnp.float32),
        scratch_types=[
            pltpu.VMEM((_COLS_PER_WORKER,), jnp.int32),
            pltpu.VMEM((_CHUNK_D, _COLS_PER_WORKER), jnp.float32),
            pltpu.VMEM((_CHUNK_D, _COLS_PER_WORKER), jnp.float32),
            pltpu.SemaphoreType.DMA,
            pltpu.SemaphoreType.DMA,
        ],
        compiler_params=pltpu.CompilerParams(
            needs_layout_passes=False,
            skip_device_barrier=True,
            disable_semaphore_checks=True,
        ),
    )(_onehot_body)
    return fn(x)


def kernel(X_in, ones):
    del ones
    return _onehot_sc(X_in.astype(jnp.int32)).T
